# baseline (speedup 1.0000x reference)
import jax
import jax.numpy as jnp
from jax.experimental import pallas as pl
from jax.experimental.pallas import tpu as pltpu

NUM_ROWS = 1000000
DIM = 32


def _gather_all_kernel(x_ref, o_ref, sem):
    copy = pltpu.make_async_copy(x_ref, o_ref, sem)
    copy.start()
    copy.wait()


def kernel(table):
    return pl.pallas_call(
        _gather_all_kernel,
        in_specs=[pl.BlockSpec(memory_space=pl.ANY)],
        out_specs=pl.BlockSpec(memory_space=pl.ANY),
        out_shape=jax.ShapeDtypeStruct((NUM_ROWS, DIM), table.dtype),
        scratch_shapes=[pltpu.SemaphoreType.DMA],
    )(table)


# speedup vs baseline: 14.1691x; 14.1691x over previous
import jax
import jax.numpy as jnp
from jax.experimental import pallas as pl
from jax.experimental.pallas import tpu as pltpu

NUM_ROWS = 1000000
DIM = 32
WIDE_ROWS = NUM_ROWS * DIM // 128
BLOCK = 2000
GRID = WIDE_ROWS // BLOCK


def _copy_kernel(x_ref, o_ref):
    o_ref[...] = x_ref[...]


def kernel(table):
    wide = table.reshape(WIDE_ROWS, 128)
    out = pl.pallas_call(
        _copy_kernel,
        grid=(GRID,),
        in_specs=[pl.BlockSpec((BLOCK, 128), lambda i: (i, 0))],
        out_specs=pl.BlockSpec((BLOCK, 128), lambda i: (i, 0)),
        out_shape=jax.ShapeDtypeStruct((WIDE_ROWS, 128), table.dtype),
    )(wide)
    return out.reshape(NUM_ROWS, DIM)


# speedup vs baseline: 14.8057x; 1.0449x over previous
import jax
import jax.numpy as jnp
from jax.experimental import pallas as pl
from jax.experimental.pallas import tpu as pltpu

NUM_ROWS = 1000000
DIM = 32
WIDE_ROWS = NUM_ROWS * DIM // 128
BLOCK = 10000
GRID = WIDE_ROWS // BLOCK


def _copy_kernel(x_ref, o_ref):
    o_ref[...] = x_ref[...]


def kernel(table):
    wide = table.reshape(WIDE_ROWS, 128)
    out = pl.pallas_call(
        _copy_kernel,
        grid=(GRID,),
        in_specs=[pl.BlockSpec((BLOCK, 128), lambda i: (i, 0))],
        out_specs=pl.BlockSpec((BLOCK, 128), lambda i: (i, 0)),
        out_shape=jax.ShapeDtypeStruct((WIDE_ROWS, 128), table.dtype),
    )(wide)
    return out.reshape(NUM_ROWS, DIM)
